# baseline (speedup 1.0000x reference)
import functools

import jax
import jax.numpy as jnp
from jax import lax
from jax.experimental import pallas as pl
from jax.experimental.pallas import tpu as pltpu
from jax.experimental.pallas import tpu_sc as plsc

NUM_CORES = 2
NUM_SUBCORES = 16
NW = NUM_CORES * NUM_SUBCORES

BATCH = 4096
HIST = 200
D = 64
TOTAL = BATCH * HIST
PER_W = TOTAL // NW
CHUNK = 128
GPS = 1
STEP_ROWS = CHUNK * GPS
STEPS = PER_W // STEP_ROWS
NCHUNK = PER_W // CHUNK
NBUF = 8

_mesh = plsc.VectorSubcoreMesh(
    core_axis_name="c", subcore_axis_name="s",
    num_cores=NUM_CORES, num_subcores=NUM_SUBCORES)


@functools.partial(
    pl.kernel,
    out_type=jax.ShapeDtypeStruct((TOTAL, D), jnp.float32),
    mesh=_mesh,
    scratch_types=[
        pltpu.VMEM((NCHUNK, CHUNK), jnp.int32),
        [pltpu.VMEM((STEP_ROWS, D), jnp.float32) for _ in range(NBUF)],
        [pltpu.SemaphoreType.DMA for _ in range(NBUF)],
        [pltpu.SemaphoreType.DMA for _ in range(NBUF)],
    ],
    compiler_params=pltpu.CompilerParams(
        use_tc_tiling_on_sc=False, skip_device_barrier=True),
)
def _gather_kernel(table_hbm, idx_hbm, out_hbm, idx_v, rows, gsem, wsem):
    wid = lax.axis_index("s") * NUM_CORES + lax.axis_index("c")
    base = wid * PER_W
    pltpu.sync_copy(idx_hbm.at[wid], idx_v)

    def fire_gather(s, b):
        for j in range(GPS):
            pltpu.make_async_copy(
                table_hbm.at[idx_v.at[s * GPS + j]],
                rows[b].at[pl.ds(j * CHUNK, CHUNK)],
                gsem[b]).start()

    def wait_gather(b):
        for j in range(GPS):
            pltpu.make_async_copy(
                table_hbm.at[idx_v.at[0]],
                rows[b].at[pl.ds(j * CHUNK, CHUNK)],
                gsem[b]).wait()

    def fire_write(s, b):
        pltpu.make_async_copy(
            rows[b], out_hbm.at[pl.ds(base + s * STEP_ROWS, STEP_ROWS)],
            wsem[b]).start()

    def wait_write(b):
        pltpu.make_async_copy(
            rows[b], out_hbm.at[pl.ds(base, STEP_ROWS)],
            wsem[b]).wait()

    for s0 in range(NBUF - 1):
        fire_gather(s0, s0)

    @pl.loop(0, STEPS, step=NBUF)
    def _grp(g):
        for k in range(NBUF):
            s = g + k
            bg = (k + NBUF - 1) % NBUF

            @pl.when(s >= 1)
            def _():
                wait_write(bg)

            @pl.when(s + NBUF - 1 < STEPS)
            def _():
                fire_gather(s + NBUF - 1, bg)

            wait_gather(k)
            fire_write(s, k)

    wait_write((STEPS - 1) % NBUF)


def kernel(input_, weight):
    idx = input_.reshape(NW, NCHUNK, CHUNK)
    out = _gather_kernel(weight, idx)
    return out.reshape(BATCH, HIST, D)


# speedup vs baseline: 1.0030x; 1.0030x over previous
import functools

import jax
import jax.numpy as jnp
from jax import lax
from jax.experimental import pallas as pl
from jax.experimental.pallas import tpu as pltpu
from jax.experimental.pallas import tpu_sc as plsc

NUM_CORES = 2
NUM_SUBCORES = 16
NW = NUM_CORES * NUM_SUBCORES

BATCH = 4096
HIST = 200
D = 64
ROWS_W = BATCH // NW
STEPS = ROWS_W
SPLIT = 120
NBUF = 4

_mesh = plsc.VectorSubcoreMesh(
    core_axis_name="c", subcore_axis_name="s",
    num_cores=NUM_CORES, num_subcores=NUM_SUBCORES)


@functools.partial(
    pl.kernel,
    out_type=jax.ShapeDtypeStruct((BATCH * HIST, D), jnp.float32),
    mesh=_mesh,
    scratch_types=[
        pltpu.VMEM((ROWS_W, HIST), jnp.int32),
        [pltpu.VMEM((HIST, D), jnp.float32) for _ in range(NBUF)],
        [pltpu.SemaphoreType.DMA for _ in range(NBUF)],
        [pltpu.SemaphoreType.DMA for _ in range(NBUF)],
    ],
    compiler_params=pltpu.CompilerParams(
        use_tc_tiling_on_sc=False, skip_device_barrier=True),
)
def _gather_kernel(table_hbm, idx_hbm, out_hbm, idx_v, rows, gsem, wsem):
    wid = lax.axis_index("s") * NUM_CORES + lax.axis_index("c")
    row0 = wid * ROWS_W
    pltpu.sync_copy(idx_hbm.at[pl.ds(row0, ROWS_W)], idx_v)

    def fire_gather(s, b):
        pltpu.make_async_copy(
            table_hbm.at[idx_v.at[s, pl.ds(0, SPLIT)]],
            rows[b].at[pl.ds(0, SPLIT)], gsem[b]).start()
        pltpu.make_async_copy(
            table_hbm.at[idx_v.at[s, pl.ds(SPLIT, HIST - SPLIT)]],
            rows[b].at[pl.ds(SPLIT, HIST - SPLIT)], gsem[b]).start()

    def wait_gather(b):
        pltpu.make_async_copy(
            table_hbm.at[idx_v.at[0, pl.ds(0, SPLIT)]],
            rows[b].at[pl.ds(0, SPLIT)], gsem[b]).wait()
        pltpu.make_async_copy(
            table_hbm.at[idx_v.at[0, pl.ds(SPLIT, HIST - SPLIT)]],
            rows[b].at[pl.ds(SPLIT, HIST - SPLIT)], gsem[b]).wait()

    def fire_write(s, b):
        pltpu.make_async_copy(
            rows[b], out_hbm.at[pl.ds((row0 + s) * HIST, HIST)],
            wsem[b]).start()

    def wait_write(b):
        pltpu.make_async_copy(
            rows[b], out_hbm.at[pl.ds(0, HIST)], wsem[b]).wait()

    for s0 in range(NBUF - 1):
        fire_gather(s0, s0)

    @pl.loop(0, STEPS, step=NBUF)
    def _grp(g):
        for k in range(NBUF):
            s = g + k
            bg = (k + NBUF - 1) % NBUF

            @pl.when(s >= 1)
            def _():
                wait_write(bg)

            @pl.when(s + NBUF - 1 < STEPS)
            def _():
                fire_gather(s + NBUF - 1, bg)

            wait_gather(k)
            fire_write(s, k)

    wait_write((STEPS - 1) % NBUF)


def kernel(input_, weight):
    out = _gather_kernel(weight, input_)
    return out.reshape(BATCH, HIST, D)


# speedup vs baseline: 1.2265x; 1.2228x over previous
import functools

import jax
import jax.numpy as jnp
from jax import lax
from jax.experimental import pallas as pl
from jax.experimental.pallas import tpu as pltpu
from jax.experimental.pallas import tpu_sc as plsc

NUM_CORES = 2
NUM_SUBCORES = 16
NW = NUM_CORES * NUM_SUBCORES

BATCH = 4096
HIST = 200
D = 64
DP = 128
ROWS_W = BATCH // NW
STEPS = 2 * ROWS_W
SPLIT = 96
SZ = (SPLIT, HIST - SPLIT)
NBUF = 4

_mesh = plsc.VectorSubcoreMesh(
    core_axis_name="c", subcore_axis_name="s",
    num_cores=NUM_CORES, num_subcores=NUM_SUBCORES)


@functools.partial(
    pl.kernel,
    out_type=jax.ShapeDtypeStruct((BATCH * HIST, DP), jnp.float32),
    mesh=_mesh,
    scratch_types=[
        pltpu.VMEM((ROWS_W, HIST), jnp.int32),
        [pltpu.VMEM((HIST - SPLIT, DP), jnp.float32) for _ in range(NBUF)],
        [pltpu.SemaphoreType.DMA for _ in range(NBUF)],
        [pltpu.SemaphoreType.DMA for _ in range(NBUF)],
    ],
    compiler_params=pltpu.CompilerParams(
        use_tc_tiling_on_sc=False, skip_device_barrier=True),
)
def _gather_kernel(table_hbm, idx_hbm, out_hbm, idx_v, rows, gsem, wsem):
    wid = lax.axis_index("s") * NUM_CORES + lax.axis_index("c")
    row0 = wid * ROWS_W
    pltpu.sync_copy(idx_hbm.at[pl.ds(row0, ROWS_W)], idx_v)

    def fire_gather(s, k, b):
        r, h = s // 2, k % 2
        pltpu.make_async_copy(
            table_hbm.at[idx_v.at[r, pl.ds(h * SPLIT, SZ[h])]],
            rows[b].at[pl.ds(0, SZ[h])], gsem[b]).start()

    def wait_gather(k, b):
        h = k % 2
        pltpu.make_async_copy(
            table_hbm.at[idx_v.at[0, pl.ds(0, SZ[h])]],
            rows[b].at[pl.ds(0, SZ[h])], gsem[b]).wait()

    def fire_write(s, k, b):
        r, h = s // 2, k % 2
        pltpu.make_async_copy(
            rows[b].at[pl.ds(0, SZ[h])],
            out_hbm.at[pl.ds((row0 + r) * HIST + h * SPLIT, SZ[h])],
            wsem[b]).start()

    def wait_write(k, b):
        h = k % 2
        pltpu.make_async_copy(
            rows[b].at[pl.ds(0, SZ[h])],
            out_hbm.at[pl.ds(0, SZ[h])], wsem[b]).wait()

    for s0 in range(NBUF - 1):
        fire_gather(s0, s0, s0)

    @pl.loop(0, STEPS, step=NBUF)
    def _grp(g):
        for k in range(NBUF):
            s = g + k
            bg = (k + NBUF - 1) % NBUF
            kg = (k + NBUF - 1) % NBUF

            @pl.when(s >= 1)
            def _():
                wait_write(kg, bg)

            @pl.when(s + NBUF - 1 < STEPS)
            def _():
                fire_gather(s + NBUF - 1, kg, bg)

            wait_gather(k, k)
            fire_write(s, k, k)

    wait_write((STEPS - 1) % NBUF, (STEPS - 1) % NBUF)


def kernel(input_, weight):
    wp = jnp.pad(weight, ((0, 0), (0, DP - D)))
    outp = _gather_kernel(wp, input_)
    return outp[:, :D].reshape(BATCH, HIST, D)
